# baseline (speedup 1.0000x reference)
import jax
import jax.numpy as jnp
from jax.experimental import pallas as pl
from jax.experimental.pallas import tpu as pltpu

_N, _C, _H, _W = 4, 96, 224, 224
_CP = 128
_WP = 256
_TROWS = 32
_TT = _TROWS * _WP
_NTF = _H // _TROWS + 2
_FLAT = _NTF * _TT
_OFF = _TT
_TILE = 2048
_CHUNK = 512
_ROWS = _TILE // _WP
_NTC = _H // _ROWS
_PH = _NTF + _NTC


def _body(x_ref, w_ref, o_ref, s_ref):
    t = pl.program_id(1)

    @pl.when(t < _NTF)
    def _transform():
        a = x_ref[0]
        at = jnp.transpose(a, (1, 0))
        pieces = [
            jnp.pad(
                jax.lax.slice(at, (h * _W, 0), ((h + 1) * _W, _C)),
                ((0, _WP - _W), (0, _CP - _C)),
            )
            for h in range(_TROWS)
        ]
        v = jnp.concatenate(pieces, axis=0)
        valid = jnp.logical_and(t >= 1, t <= _NTF - 2)
        v = jnp.where(valid, v, jnp.zeros_like(v))
        s_ref[pl.ds(jnp.minimum(t, _NTF - 1) * _TT, _TT), :] = v.astype(
            jnp.bfloat16)

    @pl.when(t >= _NTF)
    def _conv():
        base = jnp.maximum(t - _NTF, 0) * _TILE
        for c in range(_TILE // _CHUNK):
            cbase = base + c * _CHUNK
            slabs = [
                s_ref[pl.ds(cbase + _OFF - _WP - 16 + kh * _WP, _CHUNK + 24), :]
                for kh in range(3)
            ]
            cat = jnp.concatenate(slabs, axis=1)
            out = jnp.zeros((_CHUNK, _C), jnp.float32)
            for kw in range(3):
                p = jax.lax.dot_general(
                    cat, w_ref[kw],
                    (((1,), (0,)), ((), ())),
                    preferred_element_type=jnp.float32,
                )
                out = out + jax.lax.slice(
                    p, (15 + kw, 0), (15 + kw + _CHUNK, _C))
            crows = _CHUNK // _WP
            outT = jnp.transpose(out, (1, 0)).reshape(_C, crows, _WP)
            o_ref[0, :, c * crows:(c + 1) * crows, :] = jax.lax.slice(
                outT, (0, 0, 0), (_C, crows, _W))


def kernel(input, W):
    x2 = input.reshape(_N, _C, _H * _W)
    wt = jnp.transpose(W, (2, 3, 1, 0))
    wt = jnp.pad(wt, ((0, 0), (0, 0), (0, _CP - _C), (0, 0)))
    wcat = jnp.transpose(wt, (1, 0, 2, 3)).reshape(3, 3 * _CP, _C)
    wcat = wcat.astype(jnp.bfloat16)
    y = pl.pallas_call(
        _body,
        grid=(_N, _PH),
        in_specs=[
            pl.BlockSpec(
                (1, _C, _TROWS * _W),
                lambda n, t: (n, 0, jnp.clip(t - 1, 0, _H // _TROWS - 1)),
            ),
            pl.BlockSpec((3, 3 * _CP, _C), lambda n, t: (0, 0, 0)),
        ],
        out_specs=pl.BlockSpec(
            (1, _C, _ROWS, _W),
            lambda n, t: (n, 0, jnp.clip(t - _NTF, 0, _NTC - 1), 0)),
        out_shape=jax.ShapeDtypeStruct((_N, _C, _H, _W), jnp.float32),
        scratch_shapes=[pltpu.VMEM((_FLAT, _CP), jnp.bfloat16)],
    )(x2, wcat)
    return y
